# baseline (speedup 1.0000x reference)
import functools

import jax
import jax.numpy as jnp
from jax import lax
from jax.experimental import pallas as pl
from jax.experimental.pallas import tpu as pltpu
from jax.experimental.pallas import tpu_sc as plsc

_VOCAB = 128
_W = 16
_L = 16
_NC = 2
_NS = 16
_NWORKERS = _NC * _NS
_B = 16
_S = 2048
_N_WORDS = _B * _S
_WPW = _N_WORDS // _NWORKERS
_CHUNK = 128
_NCHUNK = _WPW // _CHUNK
_GROUPS = _CHUNK // _L


def _fofe_body(ff_hbm, xt_hbm, out_hbm, ff_v, chars_v, buf0, buf1, sem0, sem1):
    cid = lax.axis_index("c")
    sid = lax.axis_index("s")
    wid = sid * _NC + cid
    base_word = wid * _WPW
    b = wid // 2
    s0 = (wid % 2) * _WPW

    pltpu.sync_copy(ff_hbm, ff_v)
    pltpu.sync_copy(xt_hbm.at[b, :, pl.ds(s0, _WPW)], chars_v)

    ffv = ff_v[...]
    ones = jnp.ones((_L,), jnp.float32)
    zeros = jnp.zeros((_L,), jnp.float32)
    onesi = jnp.ones((_L,), jnp.int32)
    zerosi = jnp.zeros((_L,), jnp.int32)
    lane = lax.iota(jnp.int32, _L)
    lane_row = lane * _VOCAB
    bufs = (buf0, buf1)
    sems = (sem0, sem1)
    wrows = [jnp.full((_L,), w, jnp.int32) for w in range(_W)]
    ff2 = ffv * ffv
    ff4 = ff2 * ff2
    ff8 = ff4 * ff4
    ffpow = jnp.where((lane & 1) != 0, ffv, ones)
    ffpow = ffpow * jnp.where((lane & 2) != 0, ff2, ones)
    ffpow = ffpow * jnp.where((lane & 4) != 0, ff4, ones)
    ffpow = ffpow * jnp.where((lane & 8) != 0, ff8, ones)

    @plsc.parallel_loop(0, _CHUNK * _VOCAB // 128, unroll=4)
    def _(i):
        for j in range(8):
            buf0[pl.ds(i * 128 + j * 16, 16)] = zeros
            buf1[pl.ds(i * 128 + j * 16, 16)] = zeros

    def fill_chunk(chunk, buf):
        @plsc.parallel_loop(0, _GROUPS, unroll=4)
        def _(g):
            scol = chunk * _CHUNK + g * _L + lane
            idx0 = g * (_L * _VOCAB) + lane_row
            cnt = zerosi
            for w in range(_W - 1, -1, -1):
                ch = plsc.load_gather(chars_v, [wrows[w], scol])
                m = ch != 0
                wgt = jnp.where(m, jnp.take_along_axis(ffpow, cnt, axis=0), 0.0)
                plsc.addupdate_scatter(buf, [idx0 + ch], wgt)
                cnt = cnt + jnp.where(m, onesi, zerosi)

    def rezero_chunk(chunk, buf):
        @plsc.parallel_loop(0, _GROUPS, unroll=4)
        def _(g):
            scol = chunk * _CHUNK + g * _L + lane
            idx0 = g * (_L * _VOCAB) + lane_row
            for w in range(_W):
                ch = plsc.load_gather(chars_v, [wrows[w], scol])
                plsc.store_scatter(buf, [idx0 + ch], zeros)

    def hbm_copy(chunk, buf, sem):
        return pltpu.make_async_copy(
            buf,
            out_hbm.at[pl.ds((base_word + chunk * _CHUNK) * _VOCAB, _CHUNK * _VOCAB)],
            sem,
        )

    def outer(c2, carry):
        for bslot in range(2):
            chunk = c2 * 2 + bslot

            @pl.when(c2 >= 1)
            def _():
                hbm_copy(chunk - 2, bufs[bslot], sems[bslot]).wait()
                rezero_chunk(chunk - 2, bufs[bslot])

            fill_chunk(chunk, bufs[bslot])
            hbm_copy(chunk, bufs[bslot], sems[bslot]).start()
        return carry

    lax.fori_loop(0, _NCHUNK // 2, outer, 0)
    hbm_copy(_NCHUNK - 2, buf0, sem0).wait()
    hbm_copy(_NCHUNK - 1, buf1, sem1).wait()


@jax.jit
def _fofe_sc(ff16, xt):
    run = pl.kernel(
        _fofe_body,
        out_type=jax.ShapeDtypeStruct((_N_WORDS * _VOCAB,), jnp.float32),
        mesh=plsc.VectorSubcoreMesh(core_axis_name="c", subcore_axis_name="s"),
        compiler_params=pltpu.CompilerParams(needs_layout_passes=False),
        scratch_types=[
            pltpu.VMEM((_L,), jnp.float32),
            pltpu.VMEM((_W, _WPW), jnp.int32),
            pltpu.VMEM((_CHUNK * _VOCAB,), jnp.float32),
            pltpu.VMEM((_CHUNK * _VOCAB,), jnp.float32),
            pltpu.SemaphoreType.DMA,
            pltpu.SemaphoreType.DMA,
        ],
    )
    return run(ff16, xt)


def kernel(sents, lengths, forgetting_factor):
    B, S, Wd = sents.shape
    xt = jnp.swapaxes(sents.astype(jnp.int32), 1, 2)
    ff16 = jnp.broadcast_to(forgetting_factor.astype(jnp.float32), (_L,))
    out = _fofe_sc(ff16, xt)
    return out.reshape(B, S, _VOCAB), lengths


# speedup vs baseline: 1.0324x; 1.0324x over previous
import functools

import jax
import jax.numpy as jnp
from jax import lax
from jax.experimental import pallas as pl
from jax.experimental.pallas import tpu as pltpu
from jax.experimental.pallas import tpu_sc as plsc

_VOCAB = 128
_W = 16
_L = 16
_NC = 2
_NS = 16
_NWORKERS = _NC * _NS
_B = 16
_S = 2048
_N_WORDS = _B * _S
_WPW = _N_WORDS // _NWORKERS
_CHUNK = 128
_NCHUNK = _WPW // _CHUNK
_GROUPS = _CHUNK // _L


def _fofe_body(ff_hbm, xt_hbm, out_hbm, ff_v, chars_v, buf0, buf1, sem0, sem1):
    cid = lax.axis_index("c")
    sid = lax.axis_index("s")
    wid = sid * _NC + cid
    base_word = wid * _WPW
    b = wid // 2
    s0 = (wid % 2) * _WPW

    pltpu.sync_copy(ff_hbm, ff_v.at[pl.ds(0, 1)])
    pltpu.sync_copy(xt_hbm.at[b, :, pl.ds(s0, _WPW)], chars_v)

    zerosi0 = jnp.zeros((_L,), jnp.int32)
    ffv = jnp.take_along_axis(ff_v[...], zerosi0, axis=0)
    ones = jnp.ones((_L,), jnp.float32)
    zeros = jnp.zeros((_L,), jnp.float32)
    onesi = jnp.ones((_L,), jnp.int32)
    zerosi = jnp.zeros((_L,), jnp.int32)
    lane = lax.iota(jnp.int32, _L)
    lane_row = lane * _VOCAB
    bufs = (buf0, buf1)
    sems = (sem0, sem1)
    wrows = [jnp.full((_L,), w, jnp.int32) for w in range(_W)]
    ff2 = ffv * ffv
    ff4 = ff2 * ff2
    ff8 = ff4 * ff4
    ffpow = jnp.where((lane & 1) != 0, ffv, ones)
    ffpow = ffpow * jnp.where((lane & 2) != 0, ff2, ones)
    ffpow = ffpow * jnp.where((lane & 4) != 0, ff4, ones)
    ffpow = ffpow * jnp.where((lane & 8) != 0, ff8, ones)

    @plsc.parallel_loop(0, _CHUNK * _VOCAB // 128, unroll=4)
    def _(i):
        for j in range(8):
            buf0[pl.ds(i * 128 + j * 16, 16)] = zeros
            buf1[pl.ds(i * 128 + j * 16, 16)] = zeros

    def fill_chunk(chunk, buf):
        @plsc.parallel_loop(0, _GROUPS, unroll=4)
        def _(g):
            scol = chunk * _CHUNK + g * _L + lane
            idx0 = g * (_L * _VOCAB) + lane_row
            cnt = zerosi
            for w in range(_W - 1, -1, -1):
                ch = plsc.load_gather(chars_v, [wrows[w], scol])
                m = ch != 0
                wgt = jnp.where(m, jnp.take_along_axis(ffpow, cnt, axis=0), 0.0)
                plsc.addupdate_scatter(buf, [idx0 + ch], wgt)
                cnt = cnt + jnp.where(m, onesi, zerosi)

    def rezero_chunk(chunk, buf):
        @plsc.parallel_loop(0, _GROUPS, unroll=4)
        def _(g):
            scol = chunk * _CHUNK + g * _L + lane
            idx0 = g * (_L * _VOCAB) + lane_row
            for w in range(_W):
                ch = plsc.load_gather(chars_v, [wrows[w], scol])
                plsc.store_scatter(buf, [idx0 + ch], zeros)

    def hbm_copy(chunk, buf, sem):
        return pltpu.make_async_copy(
            buf,
            out_hbm.at[pl.ds((base_word + chunk * _CHUNK) * _VOCAB, _CHUNK * _VOCAB)],
            sem,
        )

    def outer(c2, carry):
        for bslot in range(2):
            chunk = c2 * 2 + bslot

            @pl.when(c2 >= 1)
            def _():
                hbm_copy(chunk - 2, bufs[bslot], sems[bslot]).wait()
                rezero_chunk(chunk - 2, bufs[bslot])

            fill_chunk(chunk, bufs[bslot])
            hbm_copy(chunk, bufs[bslot], sems[bslot]).start()
        return carry

    lax.fori_loop(0, _NCHUNK // 2, outer, 0)
    hbm_copy(_NCHUNK - 2, buf0, sem0).wait()
    hbm_copy(_NCHUNK - 1, buf1, sem1).wait()


@jax.jit
def _fofe_sc(ff1, xt):
    run = pl.kernel(
        _fofe_body,
        out_type=jax.ShapeDtypeStruct((_N_WORDS * _VOCAB,), jnp.float32),
        mesh=plsc.VectorSubcoreMesh(core_axis_name="c", subcore_axis_name="s"),
        compiler_params=pltpu.CompilerParams(needs_layout_passes=False),
        scratch_types=[
            pltpu.VMEM((_L,), jnp.float32),
            pltpu.VMEM((_W, _WPW), jnp.int32),
            pltpu.VMEM((_CHUNK * _VOCAB,), jnp.float32),
            pltpu.VMEM((_CHUNK * _VOCAB,), jnp.float32),
            pltpu.SemaphoreType.DMA,
            pltpu.SemaphoreType.DMA,
        ],
    )
    return run(ff1, xt)


def kernel(sents, lengths, forgetting_factor):
    B, S, Wd = sents.shape
    xt = jnp.swapaxes(sents.astype(jnp.int32), 1, 2)
    ff = forgetting_factor.astype(jnp.float32)
    out = _fofe_sc(ff, xt)
    return out.reshape(B, S, _VOCAB), lengths
